# baseline (speedup 1.0000x reference)
import functools

import jax
import jax.numpy as jnp
from jax import lax
from jax.experimental import pallas as pl
from jax.experimental.pallas import tpu as pltpu
from jax.experimental.pallas import tpu_sc as plsc

NUM_OBJ = 151
NUM_REL = 51
BATCH = 16384

NC, NS, L = 2, 16, 16
NW = NC * NS
B_PER_W = BATCH // NW
D_PAD = 128
G = 128
N_CHUNK = B_PER_W // G


V = NUM_OBJ * NUM_OBJ
PAD_RB = 2048
SLC_RB = 2048


def _pad_body(t_ref, o_ref):
    o_ref[...] = jnp.pad(t_ref[...], ((0, 0), (0, D_PAD - NUM_REL)))


@jax.jit
def _tc_pad(table):
    return pl.pallas_call(
        _pad_body,
        grid=(pl.cdiv(V, PAD_RB),),
        in_specs=[pl.BlockSpec((PAD_RB, NUM_REL), lambda i: (i, 0))],
        out_specs=pl.BlockSpec((PAD_RB, D_PAD), lambda i: (i, 0)),
        out_shape=jax.ShapeDtypeStruct((V, D_PAD), jnp.float32),
    )(table)


def _slice_body(x_ref, o_ref):
    o_ref[...] = x_ref[:, :NUM_REL]


@jax.jit
def _tc_slice(out_pad):
    return pl.pallas_call(
        _slice_body,
        grid=(BATCH // SLC_RB,),
        in_specs=[pl.BlockSpec((SLC_RB, D_PAD), lambda i: (i, 0))],
        out_specs=pl.BlockSpec((SLC_RB, NUM_REL), lambda i: (i, 0)),
        out_shape=jax.ShapeDtypeStruct((BATCH, NUM_REL), jnp.float32),
    )(out_pad)


@jax.jit
def _sc_gather(l0, l1, table_pad):
    mesh = plsc.VectorSubcoreMesh(core_axis_name="c", subcore_axis_name="s")

    @functools.partial(
        pl.kernel,
        mesh=mesh,
        out_type=jax.ShapeDtypeStruct((BATCH, D_PAD), jnp.float32),
        scratch_types=[
            pltpu.VMEM((B_PER_W,), jnp.int32),
            pltpu.VMEM((B_PER_W,), jnp.int32),
            pltpu.VMEM((B_PER_W,), jnp.int32),
            pltpu.VMEM((B_PER_W, D_PAD), jnp.float32),
            pltpu.SemaphoreType.DMA,
        ],
    )
    def k(l0_hbm, l1_hbm, table_hbm, out_hbm, l0_v, l1_v, idx_v, rows_v, sem):
        wid = lax.axis_index("s") * NC + lax.axis_index("c")
        base = wid * B_PER_W
        pltpu.sync_copy(l0_hbm.at[pl.ds(base, B_PER_W)], l0_v)
        pltpu.sync_copy(l1_hbm.at[pl.ds(base, B_PER_W)], l1_v)

        @pl.loop(0, B_PER_W, step=L)
        def _(c):
            sl = pl.ds(c, L)
            idx_v.at[sl][...] = l0_v.at[sl][...] * NUM_OBJ + l1_v.at[sl][...]

        for j in range(N_CHUNK):
            pltpu.async_copy(
                table_hbm.at[idx_v.at[pl.ds(j * G, G)]],
                rows_v.at[pl.ds(j * G, G)],
                sem,
            )
        for j in range(N_CHUNK):
            pltpu.make_async_copy(
                table_hbm.at[idx_v.at[pl.ds(j * G, G)]],
                rows_v.at[pl.ds(j * G, G)],
                sem,
            ).wait()

        pltpu.sync_copy(rows_v, out_hbm.at[pl.ds(base, B_PER_W)])

    return k(l0, l1, table_pad)


def kernel(labels, table):
    l0 = labels[:, 0].astype(jnp.int32)
    l1 = labels[:, 1].astype(jnp.int32)
    table_pad = _tc_pad(table)
    out_pad = _sc_gather(l0, l1, table_pad)
    return _tc_slice(out_pad)


# speedup vs baseline: 1.5741x; 1.5741x over previous
import functools

import jax
import jax.numpy as jnp
from jax import lax
from jax.experimental import pallas as pl
from jax.experimental.pallas import tpu as pltpu
from jax.experimental.pallas import tpu_sc as plsc
from jax.experimental import layout as jlayout

NUM_OBJ = 151
NUM_REL = 51
BATCH = 16384

NC, NS, L = 2, 16, 16
NW = NC * NS
B_PER_W = BATCH // NW
G = 128
N_CHUNK = B_PER_W // G
V = NUM_OBJ * NUM_OBJ


@jax.jit
def _sc_gather(l0, l1, table_lin):
    mesh = plsc.VectorSubcoreMesh(core_axis_name="c", subcore_axis_name="s")

    @functools.partial(
        pl.kernel,
        mesh=mesh,
        out_type=jax.ShapeDtypeStruct((BATCH, NUM_REL), jnp.float32),
        scratch_types=[
            pltpu.VMEM((B_PER_W,), jnp.int32),
            pltpu.VMEM((B_PER_W,), jnp.int32),
            pltpu.VMEM((B_PER_W,), jnp.int32),
            pltpu.VMEM((B_PER_W, NUM_REL), jnp.float32),
            pltpu.SemaphoreType.DMA,
        ],
    )
    def k(l0_hbm, l1_hbm, table_hbm, out_hbm, l0_v, l1_v, idx_v, rows_v, sem):
        wid = lax.axis_index("s") * NC + lax.axis_index("c")
        base = wid * B_PER_W
        pltpu.sync_copy(l0_hbm.at[pl.ds(base, B_PER_W)], l0_v)
        pltpu.sync_copy(l1_hbm.at[pl.ds(base, B_PER_W)], l1_v)

        @pl.loop(0, B_PER_W, step=L)
        def _(c):
            sl = pl.ds(c, L)
            idx_v.at[sl][...] = l0_v.at[sl][...] * NUM_OBJ + l1_v.at[sl][...]

        for j in range(N_CHUNK):
            pltpu.async_copy(
                table_hbm.at[idx_v.at[pl.ds(j * G, G)]],
                rows_v.at[pl.ds(j * G, G)],
                sem,
            )
        for j in range(N_CHUNK):
            pltpu.make_async_copy(
                table_hbm.at[idx_v.at[pl.ds(j * G, G)]],
                rows_v.at[pl.ds(j * G, G)],
                sem,
            ).wait()

        pltpu.sync_copy(rows_v, out_hbm.at[pl.ds(base, B_PER_W)])

    return k(l0, l1, table_lin)


def kernel(labels, table):
    l0 = labels[:, 0].astype(jnp.int32)
    l1 = labels[:, 1].astype(jnp.int32)
    table_lin = jlayout.with_layout_constraint(
        table, jlayout.Layout((0, 1), tiling=()))
    return _sc_gather(l0, l1, table_lin)


# speedup vs baseline: 1.5821x; 1.0051x over previous
import dataclasses
import functools

import jax
import jax.numpy as jnp
from jax import lax
from jax.experimental import pallas as pl
from jax.experimental.pallas import tpu as pltpu
from jax.experimental.pallas import tpu_sc as plsc
from jax.experimental import layout as jlayout

NUM_OBJ = 151
NUM_REL = 51
BATCH = 16384

NC, NS, L = 2, 16, 16
NW = NC * NS
B_PER_W = BATCH // NW
G = 128
N_CHUNK = B_PER_W // G
V = NUM_OBJ * NUM_OBJ


@jax.jit
def _sc_gather(l0, l1, table_lin):
    mesh = plsc.VectorSubcoreMesh(core_axis_name="c", subcore_axis_name="s")

    @functools.partial(
        pl.kernel,
        mesh=mesh,
        out_type=jax.ShapeDtypeStruct((BATCH, NUM_REL), jnp.float32),
        scratch_types=[
            pltpu.VMEM((B_PER_W,), jnp.int32),
            pltpu.VMEM((B_PER_W,), jnp.int32),
            pltpu.VMEM((B_PER_W,), jnp.int32),
            pltpu.VMEM((B_PER_W, NUM_REL), jnp.float32),
            pltpu.SemaphoreType.DMA((N_CHUNK,)),
            pltpu.SemaphoreType.DMA,
        ],
    )
    def k(l0_hbm, l1_hbm, table_hbm, out_hbm, l0_v, l1_v, idx_v, rows_v,
          gsem, wsem):
        wid = lax.axis_index("s") * NC + lax.axis_index("c")
        base = wid * B_PER_W
        pltpu.sync_copy(l0_hbm.at[pl.ds(base, B_PER_W)], l0_v)
        pltpu.sync_copy(l1_hbm.at[pl.ds(base, B_PER_W)], l1_v)

        for j in range(N_CHUNK):
            @pl.loop(j * G, (j + 1) * G, step=L)
            def _(c):
                sl = pl.ds(c, L)
                idx_v.at[sl][...] = l0_v.at[sl][...] * NUM_OBJ + l1_v.at[sl][...]

            pltpu.async_copy(
                table_hbm.at[idx_v.at[pl.ds(j * G, G)]],
                rows_v.at[pl.ds(j * G, G)],
                gsem.at[j],
            )
        for j in range(N_CHUNK):
            pltpu.make_async_copy(
                table_hbm.at[idx_v.at[pl.ds(j * G, G)]],
                rows_v.at[pl.ds(j * G, G)],
                gsem.at[j],
            ).wait()
            pltpu.async_copy(
                rows_v.at[pl.ds(j * G, G)],
                out_hbm.at[pl.ds(base + j * G, G)],
                wsem,
            )
        for j in range(N_CHUNK):
            pltpu.make_async_copy(
                rows_v.at[pl.ds(j * G, G)],
                out_hbm.at[pl.ds(base + j * G, G)],
                wsem,
            ).wait()

    return k(l0, l1, table_lin)


def kernel(labels, table):
    l0 = labels[:, 0].astype(jnp.int32)
    l1 = labels[:, 1].astype(jnp.int32)
    table_lin = jlayout.with_layout_constraint(
        table, jlayout.Layout((0, 1), tiling=()))
    return _sc_gather(l0, l1, table_lin)


# speedup vs baseline: 1.5843x; 1.0014x over previous
import functools

import jax
import jax.numpy as jnp
from jax import lax
from jax.experimental import pallas as pl
from jax.experimental.pallas import tpu as pltpu
from jax.experimental.pallas import tpu_sc as plsc
from jax.experimental import layout as jlayout

NUM_OBJ = 151
NUM_REL = 51
BATCH = 16384

NC, NS, L = 2, 16, 16
NW = NC * NS
B_PER_W = BATCH // NW
G = 128
N_CHUNK = B_PER_W // G
V = NUM_OBJ * NUM_OBJ


@jax.jit
def _sc_gather(l0, l1, table_lin):
    mesh = plsc.VectorSubcoreMesh(core_axis_name="c", subcore_axis_name="s")

    @functools.partial(
        pl.kernel,
        mesh=mesh,
        out_type=jax.ShapeDtypeStruct((BATCH, NUM_REL), jnp.float32),
        scratch_types=[
            pltpu.VMEM((B_PER_W,), jnp.int32),
            pltpu.VMEM((B_PER_W,), jnp.int32),
            pltpu.VMEM((B_PER_W,), jnp.int32),
            pltpu.VMEM((B_PER_W, NUM_REL), jnp.float32),
            pltpu.SemaphoreType.DMA((N_CHUNK,)),
            pltpu.SemaphoreType.DMA,
        ],
    )
    def k(l0_hbm, l1_hbm, table_hbm, out_hbm, l0_v, l1_v, idx_v, rows_v,
          gsem, wsem):
        wid = lax.axis_index("s") * NC + lax.axis_index("c")
        base = wid * B_PER_W
        pltpu.sync_copy(l0_hbm.at[pl.ds(base, B_PER_W)], l0_v)
        pltpu.sync_copy(l1_hbm.at[pl.ds(base, B_PER_W)], l1_v)

        for j in range(N_CHUNK):
            @pl.loop(j * G, (j + 1) * G, step=L)
            def _(c):
                sl = pl.ds(c, L)
                idx_v.at[sl][...] = l0_v.at[sl][...] * NUM_OBJ + l1_v.at[sl][...]

            pltpu.async_copy(
                table_hbm.at[idx_v.at[pl.ds(j * G, G)]],
                rows_v.at[pl.ds(j * G, G)],
                gsem.at[j],
            )
        for j in range(N_CHUNK):
            pltpu.make_async_copy(
                table_hbm.at[idx_v.at[pl.ds(j * G, G)]],
                rows_v.at[pl.ds(j * G, G)],
                gsem.at[j],
            ).wait()
            pltpu.async_copy(
                rows_v.at[pl.ds(j * G, G)],
                out_hbm.at[pl.ds(base + j * G, G)],
                wsem,
            )
        for j in range(N_CHUNK):
            pltpu.make_async_copy(
                rows_v.at[pl.ds(j * G, G)],
                out_hbm.at[pl.ds(base + j * G, G)],
                wsem,
            ).wait()

    return k(l0, l1, table_lin)


def kernel(labels, table):
    l0 = labels[:, 0].astype(jnp.int32)
    l1 = labels[:, 1].astype(jnp.int32)
    table_lin = jlayout.with_layout_constraint(
        table, jlayout.Layout((0, 1), tiling=()))
    return _sc_gather(l0, l1, table_lin)
